# baseline (speedup 1.0000x reference)
import functools

import jax
import jax.numpy as jnp
from jax import lax
from jax.experimental import pallas as pl
from jax.experimental.pallas import tpu as pltpu
from jax.experimental.pallas import tpu_sc as plsc

_NC, _NS, _L = 2, 16, 16
_NW = _NC * _NS
_N = 10000
_E = 320000
_D = 128
_B = 512
_T = 32
_EMB = 64
_VOCAB = 52
_NP = 10240
_EPW = _E // _NW
_CH = 80
_NCH = _EPW // _CH
_IB = 25
_NBLK = _NCH // _IB
_RPS = _NP // _NS
_CW = 128


def _make_sc_agg():
    mesh = plsc.VectorSubcoreMesh(core_axis_name="c", subcore_axis_name="s",
                                  num_cores=_NC, num_subcores=_NS)
    scratch = [
        pltpu.VMEM_SHARED((_NP, _D), jnp.float32),
        pltpu.VMEM((_IB, _CH), jnp.int32),
        pltpu.VMEM((_IB, _CH), jnp.int32),
        pltpu.VMEM((_CH, _D), jnp.float32),
        pltpu.VMEM((_CH, _D), jnp.float32),
        pltpu.VMEM((_CH, _D), jnp.float32),
        pltpu.SemaphoreType.DMA,
        pltpu.SemaphoreType.DMA,
        pltpu.SemaphoreType.DMA,
        pltpu.SemaphoreType.DMA,
        pltpu.SemaphoreType.DMA,
        pltpu.SemaphoreType.DMA,
    ]

    def body(feat, src3, dst3, z128, acc_o, acc_sh, src_v, dst_v, b0, b1, b2,
             gs0, gs1, gs2, ss0, ss1, ss2):
        cid = lax.axis_index("c")
        sid = lax.axis_index("s")
        wid = cid * _NS + sid

        @pl.loop(0, 5)
        def _(k):
            pltpu.sync_copy(z128, acc_sh.at[pl.ds(sid * _RPS + k * 128, 128)])

        plsc.subcore_barrier()

        for bi in range(_NBLK):
            pltpu.sync_copy(src3.at[wid * _NBLK + bi], src_v)
            pltpu.sync_copy(dst3.at[wid * _NBLK + bi], dst_v)
            pltpu.async_copy(feat.at[src_v.at[0]], b0, gs0)
            pltpu.async_copy(feat.at[src_v.at[1]], b1, gs1)
            pltpu.async_copy(feat.at[src_v.at[2]], b2, gs2)

            @pl.loop(0, _IB - 1, step=3)
            def _(j):
                for t, (bt, gt, st) in enumerate(
                        ((b0, gs0, ss0), (b1, gs1, ss1), (b2, gs2, ss2))):
                    pltpu.make_async_copy(feat.at[src_v.at[j + t]], bt, gt).wait()
                    pltpu.async_copy(bt, acc_sh.at[dst_v.at[j + t]], st, add=True)
                for t, (bt, gt, st) in enumerate(
                        ((b0, gs0, ss0), (b1, gs1, ss1), (b2, gs2, ss2))):
                    pltpu.make_async_copy(bt, acc_sh.at[dst_v.at[j + t]], st).wait()

                    @pl.when(j + 3 + t < _IB)
                    def _():
                        pltpu.async_copy(feat.at[src_v.at[j + 3 + t]], bt, gt)

            pltpu.make_async_copy(feat.at[src_v.at[_IB - 1]], b0, gs0).wait()
            pltpu.sync_copy(b0, acc_sh.at[dst_v.at[_IB - 1]], add=True)

        plsc.subcore_barrier()
        r0 = sid * _RPS
        pltpu.sync_copy(acc_sh.at[pl.ds(r0, _RPS)], acc_o.at[cid, pl.ds(r0, _RPS)])

    return pl.kernel(body, out_type=jax.ShapeDtypeStruct((_NC, _NP, _D), jnp.float32),
                     mesh=mesh, scratch_types=scratch)


@functools.cache
def _sc_agg():
    return _make_sc_agg()


@functools.cache
def _sc_cnt_emb():
    return _make_sc_cnt_emb()


def _make_sc_cnt_emb():
    mesh = plsc.VectorSubcoreMesh(core_axis_name="c", subcore_axis_name="s",
                                  num_cores=_NC, num_subcores=_NS)
    out_type = (jax.ShapeDtypeStruct((_NC, _NP, _CW), jnp.float32),
                jax.ShapeDtypeStruct((_B * _T, _D), jnp.float32))
    scratch = [
        pltpu.VMEM_SHARED((_NP, _CW), jnp.float32),
        pltpu.VMEM((_IB, _CH), jnp.int32),
        pltpu.VMEM((_CH, _CW), jnp.float32),
        pltpu.VMEM((128,), jnp.int32),
        pltpu.VMEM((128, _D), jnp.float32),
        pltpu.SemaphoreType.DMA,
        pltpu.SemaphoreType.DMA,
    ]

    def body(dst3, z16, ones_h, y3, etab, cnt_o, emb_o,
             cnt_sh, dst_v, ones_v, yidx_v, ebuf, sem, ssem):
        cid = lax.axis_index("c")
        sid = lax.axis_index("s")
        wid = cid * _NS + sid

        @pl.loop(0, 5)
        def _(k):
            pltpu.sync_copy(z16, cnt_sh.at[pl.ds(sid * _RPS + k * 128, 128)])

        pltpu.sync_copy(ones_h, ones_v)
        plsc.subcore_barrier()

        for bi in range(_NBLK):
            pltpu.sync_copy(dst3.at[wid * _NBLK + bi], dst_v)

            @pl.loop(0, _IB)
            def _(j):
                pltpu.async_copy(ones_v, cnt_sh.at[dst_v.at[j]], ssem, add=True)

            @pl.loop(0, _IB)
            def _(j):
                pltpu.make_async_copy(ones_v, cnt_sh.at[dst_v.at[j]], ssem).wait()

        plsc.subcore_barrier()
        r0 = sid * _RPS
        pltpu.sync_copy(cnt_sh.at[pl.ds(r0, _RPS)], cnt_o.at[cid, pl.ds(r0, _RPS)])

        @pl.loop(0, 4)
        def _(k):
            pltpu.sync_copy(y3.at[wid, k], yidx_v)
            pltpu.async_copy(etab.at[yidx_v], ebuf, sem).wait()
            pltpu.sync_copy(ebuf, emb_o.at[pl.ds(wid * 512 + k * 128, 128)])

    return pl.kernel(body, out_type=out_type, mesh=mesh, scratch_types=scratch)


_BLK = 2000


def _xr_body(xin, wrT, bl, o):
    o[:, :] = (jnp.dot(xin[:, :], wrT[:, :], preferred_element_type=jnp.float32)
               + bl[:, :])


_xr_call = pl.pallas_call(
    _xr_body,
    grid=(_N // _BLK,),
    in_specs=[
        pl.BlockSpec((_BLK, _D), lambda i: (i, 0)),
        pl.BlockSpec((_D, _D), lambda i: (0, 0)),
        pl.BlockSpec((1, _D), lambda i: (0, 0)),
    ],
    out_specs=pl.BlockSpec((_BLK, _D), lambda i: (i, 0)),
    out_shape=jax.ShapeDtypeStruct((_N, _D), jnp.float32),
)


def _sage_body(acc0, acc1, cnt0, cnt1, xwr, wlT, o):
    cnt = jnp.maximum(cnt0[:, :1] + cnt1[:, :1], 1.0)
    mean = (acc0[:, :] + acc1[:, :]) / cnt
    t = (jnp.dot(mean, wlT[:, :], preferred_element_type=jnp.float32)
         + xwr[:, :])
    o[:, :] = jnp.maximum(t, 0.0)


_sage_call = pl.pallas_call(
    _sage_body,
    grid=(_N // _BLK,),
    in_specs=[
        pl.BlockSpec((_BLK, _D), lambda i: (i, 0)),
        pl.BlockSpec((_BLK, _D), lambda i: (i, 0)),
        pl.BlockSpec((_BLK, _CW), lambda i: (i, 0)),
        pl.BlockSpec((_BLK, _CW), lambda i: (i, 0)),
        pl.BlockSpec((_BLK, _D), lambda i: (i, 0)),
        pl.BlockSpec((_D, _D), lambda i: (0, 0)),
    ],
    out_specs=pl.BlockSpec((_BLK, _D), lambda i: (i, 0)),
    out_shape=jax.ShapeDtypeStruct((_N, _D), jnp.float32),
)


def _dec_body(acc0, acc1, cnt0, cnt1, h1, batch, w2lT, w2rT, b2l,
              whT, bh, wcT, bc, emb, WihT, WhhT, bih, bhh, WoutT, bout, out):
    nblk = _N // _BLK

    def blk(i, carry):
        gsum, cntb = carry
        sl = pl.ds(i * _BLK, _BLK)
        cnt = jnp.maximum(cnt0[sl, :1] + cnt1[sl, :1], 1.0)
        mean = (acc0[sl, :] + acc1[sl, :]) / cnt
        h2 = jnp.maximum(
            jnp.dot(mean, w2lT[:, :], preferred_element_type=jnp.float32)
            + jnp.dot(h1[sl, :], w2rT[:, :], preferred_element_type=jnp.float32)
            + b2l[:, :], 0.0)
        seg = batch[i]
        oh = (lax.broadcasted_iota(jnp.int32, (_B, _BLK), 0)
              == seg[None, :]).astype(jnp.float32)
        gsum = gsum + jnp.dot(oh, h2, preferred_element_type=jnp.float32)
        cntb = cntb + jnp.sum(oh, axis=1, keepdims=True)
        return gsum, cntb

    gsum, cntb = lax.fori_loop(
        0, nblk, blk,
        (jnp.zeros((_B, _D), jnp.float32), jnp.zeros((_B, 1), jnp.float32)))
    g = gsum / jnp.maximum(cntb, 1.0)
    h = jnp.dot(g, whT[:, :], preferred_element_type=jnp.float32) + bh[:, :]
    c = jnp.dot(g, wcT[:, :], preferred_element_type=jnp.float32) + bc[:, :]
    bias = bih[:, :] + bhh[:, :]

    def step(t, hc):
        h, c = hc
        e = emb[t]
        z = (jnp.dot(e, WihT[:, :], preferred_element_type=jnp.float32)
             + jnp.dot(h, WhhT[:, :], preferred_element_type=jnp.float32)
             + bias)
        i = jax.nn.sigmoid(z[:, 0:128])
        f = jax.nn.sigmoid(z[:, 128:256])
        gg = jnp.tanh(z[:, 256:384])
        o = jax.nn.sigmoid(z[:, 384:512])
        c = f * c + i * gg
        h = o * jnp.tanh(c)
        out[t] = jnp.dot(h, WoutT[:, :], preferred_element_type=jnp.float32) + bout[:, :]
        return h, c

    lax.fori_loop(0, _T, step, (h, c))


_dec_call = pl.pallas_call(
    _dec_body,
    out_shape=jax.ShapeDtypeStruct((_T, _B, _VOCAB), jnp.float32),
)


def kernel(x, edge_index, batch, y_in, w1l, b1l, w1r, w2l, b2l, w2r,
           wh, bh, wc, bc, emb_table, W_ih, W_hh, b_ih, b_hh, Wout, bout):
    src3 = edge_index[0].astype(jnp.int32).reshape(_NW * _NBLK, _IB, _CH)
    dst3 = edge_index[1].astype(jnp.int32).reshape(_NW * _NBLK, _IB, _CH)
    y3 = jnp.transpose(y_in, (1, 0)).astype(jnp.int32).reshape(_NW, 4, 128)
    z128 = jnp.zeros((128, _D), jnp.float32)
    z16 = z128
    ones80 = jnp.ones((_CH, _CW), jnp.float32)

    etab_pad = jnp.pad(emb_table, ((0, 0), (0, _D - _EMB)))
    WihT_pad = jnp.pad(W_ih.T, ((0, _D - _EMB), (0, 0)))
    xwr = _xr_call(x, w1r.T, b1l[None, :])
    cnt, emb = _sc_cnt_emb()(dst3, z16, ones80, y3, etab_pad)
    acc1 = _sc_agg()(x, src3, dst3, z128)
    h1 = _sage_call(acc1[0], acc1[1], cnt[0], cnt[1], xwr, w1l.T)
    acc2 = _sc_agg()(h1, src3, dst3, z128)
    out = _dec_call(acc2[0], acc2[1], cnt[0], cnt[1], h1, batch.reshape(_N // _BLK, _BLK),
                    w2l.T, w2r.T, b2l[None, :], wh.T, bh[None, :],
                    wc.T, bc[None, :], emb.reshape(_T, _B, _D),
                    WihT_pad, W_hh.T, b_ih[None, :], b_hh[None, :],
                    Wout.T, bout[None, :])
    return jnp.transpose(out, (1, 0, 2))


# speedup vs baseline: 1.0716x; 1.0716x over previous
import functools

import jax
import jax.numpy as jnp
from jax import lax
from jax.experimental import pallas as pl
from jax.experimental.pallas import tpu as pltpu
from jax.experimental.pallas import tpu_sc as plsc

_NC, _NS, _L = 2, 16, 16
_NW = _NC * _NS
_N = 10000
_E = 320000
_D = 128
_B = 512
_T = 32
_EMB = 64
_VOCAB = 52
_NP = 10240
_EPW = _E // _NW
_CH = 80
_NCH = _EPW // _CH
_IB = 25
_NBLK = _NCH // _IB
_RPS = _NP // _NS
_CW = 128


def _make_sc_agg():
    mesh = plsc.VectorSubcoreMesh(core_axis_name="c", subcore_axis_name="s",
                                  num_cores=_NC, num_subcores=_NS)
    scratch = [
        pltpu.VMEM_SHARED((_NP, _D), jnp.float32),
        pltpu.VMEM((_IB, _CH), jnp.int32),
        pltpu.VMEM((_IB, _CH), jnp.int32),
        pltpu.VMEM((_CH, _D), jnp.float32),
        pltpu.VMEM((_CH, _D), jnp.float32),
        pltpu.VMEM((_CH, _D), jnp.float32),
        pltpu.SemaphoreType.DMA,
        pltpu.SemaphoreType.DMA,
        pltpu.SemaphoreType.DMA,
        pltpu.SemaphoreType.DMA,
        pltpu.SemaphoreType.DMA,
        pltpu.SemaphoreType.DMA,
    ]

    def body(feat, src3, dst3, z128, acc_o, acc_sh, src_v, dst_v, b0, b1, b2,
             gs0, gs1, gs2, ss0, ss1, ss2):
        cid = lax.axis_index("c")
        sid = lax.axis_index("s")
        wid = cid * _NS + sid

        @pl.loop(0, 5)
        def _(k):
            pltpu.sync_copy(z128, acc_sh.at[pl.ds(sid * _RPS + k * 128, 128)])

        plsc.subcore_barrier()

        for bi in range(_NBLK):
            pltpu.sync_copy(src3.at[wid * _NBLK + bi], src_v)
            pltpu.sync_copy(dst3.at[wid * _NBLK + bi], dst_v)
            pltpu.async_copy(feat.at[src_v.at[0]], b0, gs0)
            pltpu.async_copy(feat.at[src_v.at[1]], b1, gs1)
            pltpu.async_copy(feat.at[src_v.at[2]], b2, gs2)

            @pl.loop(0, _IB - 1, step=3)
            def _(j):
                for t, (bt, gt, st) in enumerate(
                        ((b0, gs0, ss0), (b1, gs1, ss1), (b2, gs2, ss2))):
                    pltpu.make_async_copy(feat.at[src_v.at[j + t]], bt, gt).wait()
                    pltpu.async_copy(bt, acc_sh.at[dst_v.at[j + t]], st, add=True)
                for t, (bt, gt, st) in enumerate(
                        ((b0, gs0, ss0), (b1, gs1, ss1), (b2, gs2, ss2))):
                    pltpu.make_async_copy(bt, acc_sh.at[dst_v.at[j + t]], st).wait()

                    @pl.when(j + 3 + t < _IB)
                    def _():
                        pltpu.async_copy(feat.at[src_v.at[j + 3 + t]], bt, gt)

            pltpu.make_async_copy(feat.at[src_v.at[_IB - 1]], b0, gs0).wait()
            pltpu.sync_copy(b0, acc_sh.at[dst_v.at[_IB - 1]], add=True)

        plsc.subcore_barrier()
        r0 = sid * _RPS
        pltpu.sync_copy(acc_sh.at[pl.ds(r0, _RPS)], acc_o.at[cid, pl.ds(r0, _RPS)])

    return pl.kernel(body, out_type=jax.ShapeDtypeStruct((_NC, _NP, _D), jnp.float32),
                     mesh=mesh, scratch_types=scratch)


@functools.cache
def _sc_agg():
    return _make_sc_agg()


@functools.cache
def _sc_cnt_emb():
    return _make_sc_cnt_emb()


def _make_sc_cnt_emb():
    mesh = plsc.VectorSubcoreMesh(core_axis_name="c", subcore_axis_name="s",
                                  num_cores=_NC, num_subcores=_NS)
    out_type = jax.ShapeDtypeStruct((_NC, _NP, _CW), jnp.float32)
    scratch = [
        pltpu.VMEM_SHARED((_NP, _CW), jnp.float32),
        pltpu.VMEM((_IB, _CH), jnp.int32),
        pltpu.VMEM((_CH, _CW), jnp.float32),
        pltpu.SemaphoreType.DMA,
    ]

    def body(dst3, z16, ones_h, cnt_o,
             cnt_sh, dst_v, ones_v, ssem):
        cid = lax.axis_index("c")
        sid = lax.axis_index("s")
        wid = cid * _NS + sid

        @pl.loop(0, 5)
        def _(k):
            pltpu.sync_copy(z16, cnt_sh.at[pl.ds(sid * _RPS + k * 128, 128)])

        pltpu.sync_copy(ones_h, ones_v)
        plsc.subcore_barrier()

        for bi in range(_NBLK):
            pltpu.sync_copy(dst3.at[wid * _NBLK + bi], dst_v)

            @pl.loop(0, _IB)
            def _(j):
                pltpu.async_copy(ones_v, cnt_sh.at[dst_v.at[j]], ssem, add=True)

            @pl.loop(0, _IB)
            def _(j):
                pltpu.make_async_copy(ones_v, cnt_sh.at[dst_v.at[j]], ssem).wait()

        plsc.subcore_barrier()
        r0 = sid * _RPS
        pltpu.sync_copy(cnt_sh.at[pl.ds(r0, _RPS)], cnt_o.at[cid, pl.ds(r0, _RPS)])

    return pl.kernel(body, out_type=out_type, mesh=mesh, scratch_types=scratch)


_BLK = 2000


def _sage_body(acc0, acc1, cnt0, cnt1, xin, wlT, wrT, bl, o):
    cnt = jnp.maximum(cnt0[:, :1] + cnt1[:, :1], 1.0)
    mean = (acc0[:, :] + acc1[:, :]) / cnt
    t = (jnp.dot(mean, wlT[:, :], preferred_element_type=jnp.float32)
         + jnp.dot(xin[:, :], wrT[:, :], preferred_element_type=jnp.float32)
         + bl[:, :])
    o[:, :] = jnp.maximum(t, 0.0)


_sage_call = pl.pallas_call(
    _sage_body,
    grid=(_N // _BLK,),
    in_specs=[
        pl.BlockSpec((_BLK, _D), lambda i: (i, 0)),
        pl.BlockSpec((_BLK, _D), lambda i: (i, 0)),
        pl.BlockSpec((_BLK, _CW), lambda i: (i, 0)),
        pl.BlockSpec((_BLK, _CW), lambda i: (i, 0)),
        pl.BlockSpec((_BLK, _D), lambda i: (i, 0)),
        pl.BlockSpec((_D, _D), lambda i: (0, 0)),
        pl.BlockSpec((_D, _D), lambda i: (0, 0)),
        pl.BlockSpec((1, _D), lambda i: (0, 0)),
    ],
    out_specs=pl.BlockSpec((_BLK, _D), lambda i: (i, 0)),
    out_shape=jax.ShapeDtypeStruct((_N, _D), jnp.float32),
)


def _dec_body(acc0, acc1, cnt0, cnt1, h1, batch, w2lT, w2rT, b2l,
              whT, bh, wcT, bc, yts, etab, WihT, WhhT, bih, bhh, WoutT, bout,
              out):
    nblk = _N // _BLK

    def blk(i, carry):
        gsum, cntb = carry
        sl = pl.ds(i * _BLK, _BLK)
        cnt = jnp.maximum(cnt0[sl, :1] + cnt1[sl, :1], 1.0)
        mean = (acc0[sl, :] + acc1[sl, :]) / cnt
        h2 = jnp.maximum(
            jnp.dot(mean, w2lT[:, :], preferred_element_type=jnp.float32)
            + jnp.dot(h1[sl, :], w2rT[:, :], preferred_element_type=jnp.float32)
            + b2l[:, :], 0.0)
        seg = batch[i]
        oh = (lax.broadcasted_iota(jnp.int32, (_B, _BLK), 0)
              == seg[None, :]).astype(jnp.float32)
        gsum = gsum + jnp.dot(oh, h2, preferred_element_type=jnp.float32)
        cntb = cntb + jnp.sum(oh, axis=1, keepdims=True)
        return gsum, cntb

    gsum, cntb = lax.fori_loop(
        0, nblk, blk,
        (jnp.zeros((_B, _D), jnp.float32), jnp.zeros((_B, 1), jnp.float32)))
    g = gsum / jnp.maximum(cntb, 1.0)
    h = jnp.dot(g, whT[:, :], preferred_element_type=jnp.float32) + bh[:, :]
    c = jnp.dot(g, wcT[:, :], preferred_element_type=jnp.float32) + bc[:, :]
    bias = bih[:, :] + bhh[:, :]
    M = jnp.dot(etab[:, :], WihT[:, :], preferred_element_type=jnp.float32)

    def step(t, hc):
        h, c = hc
        oh = (lax.broadcasted_iota(jnp.int32, (_B, _VOCAB), 1)
              == yts[t][:, None]).astype(jnp.float32)
        z = (jnp.dot(oh, M, preferred_element_type=jnp.float32)
             + jnp.dot(h, WhhT[:, :], preferred_element_type=jnp.float32)
             + bias)
        i = jax.nn.sigmoid(z[:, 0:128])
        f = jax.nn.sigmoid(z[:, 128:256])
        gg = jnp.tanh(z[:, 256:384])
        o = jax.nn.sigmoid(z[:, 384:512])
        c = f * c + i * gg
        h = o * jnp.tanh(c)
        out[t] = jnp.dot(h, WoutT[:, :], preferred_element_type=jnp.float32) + bout[:, :]
        return h, c

    lax.fori_loop(0, _T, step, (h, c))


_dec_call = pl.pallas_call(
    _dec_body,
    out_shape=jax.ShapeDtypeStruct((_T, _B, _VOCAB), jnp.float32),
)


def kernel(x, edge_index, batch, y_in, w1l, b1l, w1r, w2l, b2l, w2r,
           wh, bh, wc, bc, emb_table, W_ih, W_hh, b_ih, b_hh, Wout, bout):
    src3 = edge_index[0].astype(jnp.int32).reshape(_NW * _NBLK, _IB, _CH)
    dst3 = edge_index[1].astype(jnp.int32).reshape(_NW * _NBLK, _IB, _CH)
    yts = jnp.transpose(y_in, (1, 0)).astype(jnp.int32)
    z128 = jnp.zeros((128, _D), jnp.float32)
    z16 = z128
    ones80 = jnp.ones((_CH, _CW), jnp.float32)

    cnt = _sc_cnt_emb()(dst3, z16, ones80)
    acc1 = _sc_agg()(x, src3, dst3, z128)
    h1 = _sage_call(acc1[0], acc1[1], cnt[0], cnt[1], x,
                    w1l.T, w1r.T, b1l[None, :])
    acc2 = _sc_agg()(h1, src3, dst3, z128)
    out = _dec_call(acc2[0], acc2[1], cnt[0], cnt[1], h1, batch.reshape(_N // _BLK, _BLK),
                    w2l.T, w2r.T, b2l[None, :], wh.T, bh[None, :],
                    wc.T, bc[None, :], yts, emb_table, W_ih.T,
                    W_hh.T, b_ih[None, :], b_hh[None, :],
                    Wout.T, bout[None, :])
    return jnp.transpose(out, (1, 0, 2))
